# baseline (speedup 1.0000x reference)
import functools

import jax
import jax.numpy as jnp
from jax import lax
from jax.experimental import pallas as pl
from jax.experimental.pallas import tpu as pltpu
from jax.experimental.pallas import tpu_sc as plsc

_BB = 16
_RING = 4
_AHEAD = 2


def kernel(x, table):
    B, H = x.shape
    V, D = table.shape
    NB = B * H

    info = plsc.get_sparse_core_info()
    NC, NS = info.num_cores, info.num_subcores
    NW = NC * NS

    C = _BB * H
    b_per_w = B // NW
    n_chunks = b_per_w // _BB
    n_outer = n_chunks // _RING

    x_flat = x.reshape(NW, n_chunks, C).astype(jnp.int32)
    mesh = plsc.VectorSubcoreMesh(core_axis_name="c", subcore_axis_name="s")

    @functools.partial(
        pl.kernel,
        mesh=mesh,
        out_type=jax.ShapeDtypeStruct((B, H, D), jnp.float32),
        scratch_types=[
            pltpu.VMEM((n_chunks, C), jnp.int32),
            pltpu.VMEM((_RING, C, D), jnp.float32),
            pltpu.SemaphoreType.DMA((_RING,)),
            pltpu.SemaphoreType.DMA((_RING,)),
        ],
        compiler_params=pltpu.CompilerParams(use_tc_tiling_on_sc=False),
    )
    def gather_kernel(x_hbm, table_hbm, out_hbm, idx_v, rows_v, gsem, ssem):
        wid = lax.axis_index("s") * NC + lax.axis_index("c")
        bbase = wid * b_per_w
        pltpu.sync_copy(x_hbm.at[wid], idx_v)

        def start_gather(j, b):
            pltpu.async_copy(table_hbm.at[idx_v.at[j]], rows_v.at[b], gsem.at[b])

        def wait_gather(j, b):
            pltpu.make_async_copy(
                table_hbm.at[idx_v.at[j]], rows_v.at[b], gsem.at[b]
            ).wait()

        def start_store(j, b):
            for k in range(_BB):
                pltpu.async_copy(
                    rows_v.at[b, pl.ds(k * H, H)],
                    out_hbm.at[bbase + j * _BB + k],
                    ssem.at[b],
                )

        def wait_store(j, b):
            for k in range(_BB):
                pltpu.make_async_copy(
                    rows_v.at[b, pl.ds(k * H, H)],
                    out_hbm.at[bbase + j * _BB + k],
                    ssem.at[b],
                ).wait()

        for u in range(_AHEAD):
            start_gather(u, u)

        def outer(it, carry):
            j0 = it * _RING
            for u in range(_RING):
                j = j0 + u
                wait_gather(j, u)
                start_store(j, u)
                ji = j + _AHEAD
                bi = (u + _AHEAD) % _RING

                @pl.when(ji < n_chunks)
                def _():
                    @pl.when(ji >= _RING)
                    def _():
                        wait_store(ji - _RING, bi)

                    start_gather(ji, bi)

            return carry

        lax.fori_loop(0, n_outer, outer, 0)

        for u in range(_RING):
            wait_store(n_chunks - _RING + u, u)

    return gather_kernel(x_flat, table)


# speedup vs baseline: 1.0325x; 1.0325x over previous
import functools

import jax
import jax.numpy as jnp
from jax import lax
from jax.experimental import pallas as pl
from jax.experimental.pallas import tpu as pltpu
from jax.experimental.pallas import tpu_sc as plsc

_BB = 8
_RING = 4
_AHEAD = 2
_DPAD = 128


def kernel(x, table):
    B, H = x.shape
    V, D = table.shape
    NB = B * H

    info = plsc.get_sparse_core_info()
    NC, NS = info.num_cores, info.num_subcores
    NW = NC * NS

    C = _BB * H
    b_per_w = B // NW
    n_chunks = b_per_w // _BB
    n_outer = n_chunks // _RING

    x_flat = x.reshape(NW, n_chunks, C).astype(jnp.int32)
    table_pad = jnp.pad(table, ((0, 0), (0, _DPAD - D)))
    mesh = plsc.VectorSubcoreMesh(core_axis_name="c", subcore_axis_name="s")

    @functools.partial(
        pl.kernel,
        mesh=mesh,
        out_type=jax.ShapeDtypeStruct((B, H, D), jnp.float32),
        scratch_types=[
            pltpu.VMEM((n_chunks, C), jnp.int32),
            pltpu.VMEM((_RING, C, _DPAD), jnp.float32),
            pltpu.SemaphoreType.DMA((_RING,)),
            pltpu.SemaphoreType.DMA((_RING,)),
        ],
        compiler_params=pltpu.CompilerParams(use_tc_tiling_on_sc=False),
    )
    def gather_kernel(x_hbm, table_hbm, out_hbm, idx_v, rows_v, gsem, ssem):
        wid = lax.axis_index("s") * NC + lax.axis_index("c")
        bbase = wid * b_per_w
        pltpu.sync_copy(x_hbm.at[wid], idx_v)

        def start_gather(j, b):
            pltpu.async_copy(table_hbm.at[idx_v.at[j]], rows_v.at[b], gsem.at[b])

        def wait_gather(j, b):
            pltpu.make_async_copy(
                table_hbm.at[idx_v.at[j]], rows_v.at[b], gsem.at[b]
            ).wait()

        def start_store(j, b):
            for k in range(_BB):
                pltpu.async_copy(
                    rows_v.at[b, pl.ds(k * H, H), pl.ds(0, D)],
                    out_hbm.at[bbase + j * _BB + k],
                    ssem.at[b],
                )

        def wait_store(j, b):
            for k in range(_BB):
                pltpu.make_async_copy(
                    rows_v.at[b, pl.ds(k * H, H), pl.ds(0, D)],
                    out_hbm.at[bbase + j * _BB + k],
                    ssem.at[b],
                ).wait()

        for u in range(_AHEAD):
            start_gather(u, u)

        def outer(it, carry):
            j0 = it * _RING
            for u in range(_RING):
                j = j0 + u
                wait_gather(j, u)
                start_store(j, u)
                ji = j + _AHEAD
                bi = (u + _AHEAD) % _RING

                @pl.when(ji < n_chunks)
                def _():
                    @pl.when(ji >= _RING)
                    def _():
                        wait_store(ji - _RING, bi)

                    start_gather(ji, bi)

            return carry

        lax.fori_loop(0, n_outer, outer, 0)

        for u in range(_RING):
            wait_store(n_chunks - _RING + u, u)

    return gather_kernel(x_flat, table_pad)


# speedup vs baseline: 1.1997x; 1.1620x over previous
import functools

import jax
import jax.numpy as jnp
from jax import lax
from jax.experimental import pallas as pl
from jax.experimental.pallas import tpu as pltpu
from jax.experimental.pallas import tpu_sc as plsc

_BB = 8
_RING = 4
_AHEAD = 2
_DPAD = 128


def kernel(x, table):
    B, H = x.shape
    V, D = table.shape
    NB = B * H

    info = plsc.get_sparse_core_info()
    NC, NS = info.num_cores, info.num_subcores
    NW = NC * NS

    C = _BB * H
    b_per_w = B // NW
    n_chunks = b_per_w // _BB
    n_outer = n_chunks // _RING

    x_flat = x.reshape(NW, n_chunks, C).astype(jnp.int32)
    table_pad = jnp.pad(table, ((0, 0), (0, _DPAD - D)))
    mesh = plsc.VectorSubcoreMesh(core_axis_name="c", subcore_axis_name="s")

    HP = 24

    @functools.partial(
        pl.kernel,
        mesh=mesh,
        out_type=jax.ShapeDtypeStruct((B, HP, _DPAD), jnp.float32),
        scratch_types=[
            pltpu.VMEM((n_chunks, C), jnp.int32),
            pltpu.VMEM((_RING, C, _DPAD), jnp.float32),
            pltpu.SemaphoreType.DMA((_RING,)),
            pltpu.SemaphoreType.DMA((_RING,)),
        ],
        compiler_params=pltpu.CompilerParams(use_tc_tiling_on_sc=False),
    )
    def gather_kernel(x_hbm, table_hbm, out_hbm, idx_v, rows_v, gsem, ssem):
        wid = lax.axis_index("s") * NC + lax.axis_index("c")
        bbase = wid * b_per_w
        pltpu.sync_copy(x_hbm.at[wid], idx_v)

        def start_gather(j, b):
            pltpu.async_copy(table_hbm.at[idx_v.at[j]], rows_v.at[b], gsem.at[b])

        def wait_gather(j, b):
            pltpu.make_async_copy(
                table_hbm.at[idx_v.at[j]], rows_v.at[b], gsem.at[b]
            ).wait()

        def start_store(j, b):
            for k in range(_BB):
                pltpu.async_copy(
                    rows_v.at[b, pl.ds(k * H, H)],
                    out_hbm.at[bbase + j * _BB + k, pl.ds(0, H)],
                    ssem.at[b],
                )

        def wait_store(j, b):
            for k in range(_BB):
                pltpu.make_async_copy(
                    rows_v.at[b, pl.ds(k * H, H)],
                    out_hbm.at[bbase + j * _BB + k, pl.ds(0, H)],
                    ssem.at[b],
                ).wait()

        for u in range(_AHEAD):
            start_gather(u, u)

        def outer(it, carry):
            j0 = it * _RING
            for u in range(_RING):
                j = j0 + u
                wait_gather(j, u)
                start_store(j, u)
                ji = j + _AHEAD
                bi = (u + _AHEAD) % _RING

                @pl.when(ji < n_chunks)
                def _():
                    @pl.when(ji >= _RING)
                    def _():
                        wait_store(ji - _RING, bi)

                    start_gather(ji, bi)

            return carry

        lax.fori_loop(0, n_outer, outer, 0)

        for u in range(_RING):
            wait_store(n_chunks - _RING + u, u)

    out_pad = gather_kernel(x_flat, table_pad)
    return out_pad[:, :H, :D]


# speedup vs baseline: 1.2995x; 1.0831x over previous
import functools

import jax
import jax.numpy as jnp
from jax import lax
from jax.experimental import pallas as pl
from jax.experimental.pallas import tpu as pltpu
from jax.experimental.pallas import tpu_sc as plsc

_BB = 8
_RING = 4
_AHEAD = 2
_DPAD = 128


def kernel(x, table):
    B, H = x.shape
    V, D = table.shape
    NB = B * H

    info = plsc.get_sparse_core_info()
    NC, NS = info.num_cores, info.num_subcores
    NW = NC * NS

    C = _BB * H
    b_per_w = B // NW
    n_chunks = b_per_w // _BB
    n_outer = n_chunks // _RING

    x_flat = (x.reshape(NW, n_chunks, C) * 2).astype(jnp.int32)
    table_pad = jnp.pad(table, ((0, 0), (0, _DPAD - D))).reshape(2 * V, D)
    mesh = plsc.VectorSubcoreMesh(core_axis_name="c", subcore_axis_name="s")

    HP = 24

    @functools.partial(
        pl.kernel,
        mesh=mesh,
        out_type=jax.ShapeDtypeStruct((B, HP, _DPAD), jnp.float32),
        scratch_types=[
            pltpu.VMEM((n_chunks, C), jnp.int32),
            pltpu.VMEM((_RING, C, D), jnp.float32),
            pltpu.SemaphoreType.DMA((_RING,)),
            pltpu.SemaphoreType.DMA((_RING,)),
        ],
        compiler_params=pltpu.CompilerParams(use_tc_tiling_on_sc=False),
    )
    def gather_kernel(x_hbm, table_hbm, out_hbm, idx_v, rows_v, gsem, ssem):
        wid = lax.axis_index("s") * NC + lax.axis_index("c")
        bbase = wid * b_per_w
        pltpu.sync_copy(x_hbm.at[wid], idx_v)

        def start_gather(j, b):
            pltpu.async_copy(table_hbm.at[idx_v.at[j]], rows_v.at[b], gsem.at[b])

        def wait_gather(j, b):
            pltpu.make_async_copy(
                table_hbm.at[idx_v.at[j]], rows_v.at[b], gsem.at[b]
            ).wait()

        def start_store(j, b):
            for k in range(_BB):
                pltpu.async_copy(
                    rows_v.at[b, pl.ds(k * H, H)],
                    out_hbm.at[bbase + j * _BB + k, pl.ds(0, H), pl.ds(0, D)],
                    ssem.at[b],
                )

        def wait_store(j, b):
            for k in range(_BB):
                pltpu.make_async_copy(
                    rows_v.at[b, pl.ds(k * H, H)],
                    out_hbm.at[bbase + j * _BB + k, pl.ds(0, H), pl.ds(0, D)],
                    ssem.at[b],
                ).wait()

        for u in range(_AHEAD):
            start_gather(u, u)

        def outer(it, carry):
            j0 = it * _RING
            for u in range(_RING):
                j = j0 + u
                wait_gather(j, u)
                start_store(j, u)
                ji = j + _AHEAD
                bi = (u + _AHEAD) % _RING

                @pl.when(ji < n_chunks)
                def _():
                    @pl.when(ji >= _RING)
                    def _():
                        wait_store(ji - _RING, bi)

                    start_gather(ji, bi)

            return carry

        lax.fori_loop(0, n_outer, outer, 0)

        for u in range(_RING):
            wait_store(n_chunks - _RING + u, u)

    out_pad = gather_kernel(x_flat, table_pad)
    return out_pad[:, :H, :D]


# speedup vs baseline: 1.3001x; 1.0005x over previous
import functools

import jax
import jax.numpy as jnp
from jax import lax
from jax.experimental import pallas as pl
from jax.experimental.pallas import tpu as pltpu
from jax.experimental.pallas import tpu_sc as plsc

_BB = 16
_RING = 4
_AHEAD = 2
_DPAD = 128


def kernel(x, table):
    B, H = x.shape
    V, D = table.shape
    NB = B * H

    info = plsc.get_sparse_core_info()
    NC, NS = info.num_cores, info.num_subcores
    NW = NC * NS

    C = _BB * H
    b_per_w = B // NW
    n_chunks = b_per_w // _BB
    n_outer = n_chunks // _RING

    x_flat = (x.reshape(NW, n_chunks, C) * 2).astype(jnp.int32)
    table_pad = jnp.pad(table, ((0, 0), (0, _DPAD - D))).reshape(2 * V, D)
    mesh = plsc.VectorSubcoreMesh(core_axis_name="c", subcore_axis_name="s")

    HP = 24

    @functools.partial(
        pl.kernel,
        mesh=mesh,
        out_type=jax.ShapeDtypeStruct((B, HP, _DPAD), jnp.float32),
        scratch_types=[
            pltpu.VMEM((n_chunks, C), jnp.int32),
            pltpu.VMEM((_RING, C, D), jnp.float32),
            pltpu.SemaphoreType.DMA((_RING,)),
            pltpu.SemaphoreType.DMA((_RING,)),
        ],
        compiler_params=pltpu.CompilerParams(use_tc_tiling_on_sc=False),
    )
    def gather_kernel(x_hbm, table_hbm, out_hbm, idx_v, rows_v, gsem, ssem):
        wid = lax.axis_index("s") * NC + lax.axis_index("c")
        bbase = wid * b_per_w
        pltpu.sync_copy(x_hbm.at[wid], idx_v)

        def start_gather(j, b):
            pltpu.async_copy(table_hbm.at[idx_v.at[j]], rows_v.at[b], gsem.at[b])

        def wait_gather(j, b):
            pltpu.make_async_copy(
                table_hbm.at[idx_v.at[j]], rows_v.at[b], gsem.at[b]
            ).wait()

        def start_store(j, b):
            for k in range(_BB):
                pltpu.async_copy(
                    rows_v.at[b, pl.ds(k * H, H)],
                    out_hbm.at[bbase + j * _BB + k, pl.ds(0, H), pl.ds(0, D)],
                    ssem.at[b],
                )

        def wait_store(j, b):
            for k in range(_BB):
                pltpu.make_async_copy(
                    rows_v.at[b, pl.ds(k * H, H)],
                    out_hbm.at[bbase + j * _BB + k, pl.ds(0, H), pl.ds(0, D)],
                    ssem.at[b],
                ).wait()

        for u in range(_AHEAD):
            start_gather(u, u)

        def outer(it, carry):
            j0 = it * _RING
            for u in range(_RING):
                j = j0 + u
                wait_gather(j, u)
                start_store(j, u)
                ji = j + _AHEAD
                bi = (u + _AHEAD) % _RING

                @pl.when(ji < n_chunks)
                def _():
                    @pl.when(ji >= _RING)
                    def _():
                        wait_store(ji - _RING, bi)

                    start_gather(ji, bi)

            return carry

        lax.fori_loop(0, n_outer, outer, 0)

        for u in range(_RING):
            wait_store(n_chunks - _RING + u, u)

    out_pad = gather_kernel(x_flat, table_pad)
    return out_pad[:, :H, :D]
